# baseline (speedup 1.0000x reference)
import jax
import jax.numpy as jnp
from jax import lax
from jax.experimental import pallas as pl
from jax.experimental.pallas import tpu as pltpu
from jax.experimental.pallas import tpu_sc as plsc

BATCH = 16384
WIDTH = 64
NW = 32
ROWS_PER_W = BATCH // NW
GCHUNK = 128
NCH = ROWS_PER_W // GCHUNK
GRP_PER_CH = GCHUNK // 16
EPS2 = 1e-16


def _body(hidT_hbm, pid_hbm, rel_hbm, proto2_hbm, out_hbm,
          pid_v, rel_v, hid_v, rows_v, out_v, sem_h, *sems):
    cid = lax.axis_index("c")
    sid = lax.axis_index("s")
    wid = sid * 2 + cid
    base = wid * ROWS_PER_W

    pltpu.sync_copy(pid_hbm.at[pl.ds(base, ROWS_PER_W)], pid_v)
    pltpu.sync_copy(rel_hbm.at[pl.ds(base, ROWS_PER_W)], rel_v)
    hcopy = pltpu.async_copy(hidT_hbm.at[:, pl.ds(base, ROWS_PER_W)],
                             hid_v, sem_h)
    gcopies = [
        pltpu.async_copy(proto2_hbm.at[pid_v.at[pl.ds(j * GCHUNK, GCHUNK)]],
                         rows_v.at[pl.ds(j * GCHUNK, GCHUNK)], sems[j])
        for j in range(NCH)
    ]
    hcopy.wait()

    iota = lax.iota(jnp.int32, 16)
    eps2 = jnp.full((16,), EPS2, jnp.float32)
    zero = jnp.zeros((16,), jnp.float32)
    one_i = jnp.full((16,), 1, jnp.int32)
    thirteen = jnp.full((16,), 13, jnp.int32)
    six = jnp.full((16,), 6, jnp.int32)
    magic = jnp.full((16,), 0x5F3759DF, jnp.int32)
    c15 = jnp.full((16,), 1.5, jnp.float32)
    half = jnp.full((16,), 0.5, jnp.float32)

    def group(g):
        row0 = g * 16
        rows16 = iota + row0
        rel16 = rel_v[pl.ds(row0, 16)]
        col = lax.shift_left(
            lax.shift_right_logical(rel16, thirteen) & one_i, six)
        d = [zero, zero, zero, zero]
        h = [zero, zero, zero, zero]
        p = [zero, zero, zero, zero]
        for c in range(WIDTH):
            vh = hid_v[c, pl.ds(row0, 16)]
            vp = plsc.load_gather(rows_v, [rows16, col])
            col = col + one_i
            a = c % 4
            d[a] = d[a] + vh * vp
            h[a] = h[a] + vh * vh
            p[a] = p[a] + vp * vp
        dot = (d[0] + d[1]) + (d[2] + d[3])
        hh = (h[0] + h[1]) + (h[2] + h[3])
        pp = (p[0] + p[1]) + (p[2] + p[3])
        d2 = jnp.maximum(hh, eps2) * jnp.maximum(pp, eps2)
        i = magic - lax.shift_right_logical(plsc.bitcast(d2, jnp.int32), one_i)
        y = plsc.bitcast(i, jnp.float32)
        for _ in range(3):
            y = y * (c15 - half * d2 * y * y)
        out_v[pl.ds(row0, 16)] = dot * y

    for j in range(NCH):
        gcopies[j].wait()
        plsc.parallel_loop(j * GRP_PER_CH, (j + 1) * GRP_PER_CH, 1,
                           unroll=2)(group)

    pltpu.sync_copy(out_v, out_hbm.at[pl.ds(base, ROWS_PER_W)])


_TBLK = 16384
_TGRID = 7
_PROWS = _TGRID * (_TBLK // 2)


def _pair_table_body(a_ref, o_ref):
    t = a_ref[...].T
    o_ref[:, 0:WIDTH] = t[0:_TBLK // 2, :]
    o_ref[:, WIDTH:2 * WIDTH] = t[_TBLK // 2:_TBLK, :]


def _pair_table(protoT):
    return pl.pallas_call(
        _pair_table_body,
        grid=(_TGRID,),
        in_specs=[pl.BlockSpec((WIDTH, _TBLK), lambda i: (0, i))],
        out_specs=pl.BlockSpec((_TBLK // 2, 2 * WIDTH), lambda i: (i, 0)),
        out_shape=jax.ShapeDtypeStruct((_PROWS, 2 * WIDTH), jnp.float32),
        compiler_params=pltpu.CompilerParams(
            fuse_transposed_lhs_in_matmul=True),
    )(protoT)


def kernel(hidden, rel_ids, prototypes):
    rel = rel_ids.astype(jnp.int32)
    pid = (lax.shift_right_logical(rel, 14) * (_TBLK // 2)
           + (rel & (_TBLK // 2 - 1)))
    proto2 = _pair_table(prototypes.T)
    hidT = hidden.T
    mesh = plsc.VectorSubcoreMesh(core_axis_name="c", subcore_axis_name="s")
    f = pl.kernel(
        _body,
        mesh=mesh,
        out_type=jax.ShapeDtypeStruct((BATCH,), jnp.float32),
        scratch_types=[
            pltpu.VMEM((ROWS_PER_W,), jnp.int32),
            pltpu.VMEM((ROWS_PER_W,), jnp.int32),
            pltpu.VMEM((WIDTH, ROWS_PER_W), jnp.float32),
            pltpu.VMEM((ROWS_PER_W, 2 * WIDTH), jnp.float32),
            pltpu.VMEM((ROWS_PER_W,), jnp.float32),
            pltpu.SemaphoreType.DMA,
            pltpu.SemaphoreType.DMA,
            pltpu.SemaphoreType.DMA,
            pltpu.SemaphoreType.DMA,
            pltpu.SemaphoreType.DMA,
        ],
        compiler_params=pltpu.CompilerParams(
            needs_layout_passes=False, use_tc_tiling_on_sc=True),
    )
    return f(hidT, pid, rel, proto2)


# speedup vs baseline: 1.0052x; 1.0052x over previous
import jax
import jax.numpy as jnp
from jax import lax
from jax.experimental import pallas as pl
from jax.experimental.pallas import tpu as pltpu
from jax.experimental.pallas import tpu_sc as plsc

BATCH = 16384
WIDTH = 64
NW = 32
ROWS_PER_W = BATCH // NW
GCHUNK = 128
NCH = ROWS_PER_W // GCHUNK
GRP_PER_CH = GCHUNK // 16
EPS2 = 1e-16


def _body(hidT_hbm, pid_hbm, rel_hbm, proto2_hbm, out_hbm,
          pid_v, rel_v, hid_v, rows_v, out_v, sem_h, *sems):
    cid = lax.axis_index("c")
    sid = lax.axis_index("s")
    wid = sid * 2 + cid
    base = wid * ROWS_PER_W

    pltpu.sync_copy(pid_hbm.at[pl.ds(base, ROWS_PER_W)], pid_v)
    pltpu.sync_copy(rel_hbm.at[pl.ds(base, ROWS_PER_W)], rel_v)
    hcopy = pltpu.async_copy(hidT_hbm.at[:, pl.ds(base, ROWS_PER_W)],
                             hid_v, sem_h)
    gcopies = [
        pltpu.async_copy(proto2_hbm.at[pid_v.at[pl.ds(j * GCHUNK, GCHUNK)]],
                         rows_v.at[pl.ds(j * GCHUNK, GCHUNK)], sems[j])
        for j in range(NCH)
    ]
    hcopy.wait()

    iota = lax.iota(jnp.int32, 16)
    eps2 = jnp.full((16,), EPS2, jnp.float32)
    zero = jnp.zeros((16,), jnp.float32)
    one_i = jnp.full((16,), 1, jnp.int32)
    twelve = jnp.full((16,), 12, jnp.int32)
    six = jnp.full((16,), 6, jnp.int32)
    magic = jnp.full((16,), 0x5F3759DF, jnp.int32)
    c15 = jnp.full((16,), 1.5, jnp.float32)
    half = jnp.full((16,), 0.5, jnp.float32)

    def group(g):
        row0 = g * 16
        rows16 = iota + row0
        rel16 = rel_v[pl.ds(row0, 16)]
        col = lax.shift_left(
            lax.shift_right_logical(rel16, twelve) & one_i, six)
        d = [zero, zero, zero, zero]
        h = [zero, zero, zero, zero]
        p = [zero, zero, zero, zero]
        for c in range(WIDTH):
            vh = hid_v[c, pl.ds(row0, 16)]
            vp = plsc.load_gather(rows_v, [rows16, col])
            col = col + one_i
            a = c % 4
            d[a] = d[a] + vh * vp
            h[a] = h[a] + vh * vh
            p[a] = p[a] + vp * vp
        dot = (d[0] + d[1]) + (d[2] + d[3])
        hh = (h[0] + h[1]) + (h[2] + h[3])
        pp = (p[0] + p[1]) + (p[2] + p[3])
        d2 = jnp.maximum(hh, eps2) * jnp.maximum(pp, eps2)
        i = magic - lax.shift_right_logical(plsc.bitcast(d2, jnp.int32), one_i)
        y = plsc.bitcast(i, jnp.float32)
        for _ in range(3):
            y = y * (c15 - half * d2 * y * y)
        out_v[pl.ds(row0, 16)] = dot * y

    for j in range(NCH):
        gcopies[j].wait()
        plsc.parallel_loop(j * GRP_PER_CH, (j + 1) * GRP_PER_CH, 1,
                           unroll=2)(group)

    pltpu.sync_copy(out_v, out_hbm.at[pl.ds(base, ROWS_PER_W)])


_TBLK = 8192
_TGRID = 13
_PROWS = _TGRID * (_TBLK // 2)


def _pair_table_body(a_ref, o_ref):
    t = a_ref[...].T
    o_ref[:, 0:WIDTH] = t[0:_TBLK // 2, :]
    o_ref[:, WIDTH:2 * WIDTH] = t[_TBLK // 2:_TBLK, :]


def _pair_table(protoT):
    return pl.pallas_call(
        _pair_table_body,
        grid=(_TGRID,),
        in_specs=[pl.BlockSpec((WIDTH, _TBLK), lambda i: (0, i))],
        out_specs=pl.BlockSpec((_TBLK // 2, 2 * WIDTH), lambda i: (i, 0)),
        out_shape=jax.ShapeDtypeStruct((_PROWS, 2 * WIDTH), jnp.float32),
        compiler_params=pltpu.CompilerParams(
            fuse_transposed_lhs_in_matmul=True),
    )(protoT)


def kernel(hidden, rel_ids, prototypes):
    rel = rel_ids.astype(jnp.int32)
    pid = (lax.shift_right_logical(rel, 13) * (_TBLK // 2)
           + (rel & (_TBLK // 2 - 1)))
    proto2 = _pair_table(prototypes.T)
    hidT = hidden.T
    mesh = plsc.VectorSubcoreMesh(core_axis_name="c", subcore_axis_name="s")
    f = pl.kernel(
        _body,
        mesh=mesh,
        out_type=jax.ShapeDtypeStruct((BATCH,), jnp.float32),
        scratch_types=[
            pltpu.VMEM((ROWS_PER_W,), jnp.int32),
            pltpu.VMEM((ROWS_PER_W,), jnp.int32),
            pltpu.VMEM((WIDTH, ROWS_PER_W), jnp.float32),
            pltpu.VMEM((ROWS_PER_W, 2 * WIDTH), jnp.float32),
            pltpu.VMEM((ROWS_PER_W,), jnp.float32),
            pltpu.SemaphoreType.DMA,
            pltpu.SemaphoreType.DMA,
            pltpu.SemaphoreType.DMA,
            pltpu.SemaphoreType.DMA,
            pltpu.SemaphoreType.DMA,
        ],
        compiler_params=pltpu.CompilerParams(
            needs_layout_passes=False, use_tc_tiling_on_sc=True),
    )
    return f(hidT, pid, rel, proto2)
